# baseline (speedup 1.0000x reference)
import jax
import jax.numpy as jnp
from jax import lax
from jax.experimental import pallas as pl
from jax.experimental.pallas import tpu as pltpu
from jax.experimental.pallas import tpu_sc as plsc

N_NODES = 10000
E_EDGES = 320000
CH = 128

NC = 2
NS = 16
NW = NC * NS
EPW = E_EDGES // NW
RPT = N_NODES // NS
WB = 624
WB_LAST = N_NODES - 15 * WB
DDEG = 8

CHK_D = 128
EPAD = 10112
NCHK_D = EPAD // CHK_D
DUMP = N_NODES
NDEG = N_NODES + 8

RBLK = 1000
NBLK = N_NODES // RBLK

DEG_W = 8


def _deg_body(dst1d, zerosd, ones8, degp, ones_v, dsts_v, deg_sh, sem):
    cid = lax.axis_index("c")
    sid = lax.axis_index("s")
    wid = sid * NC + cid

    @pl.when(sid < NS - 1)
    def _():
        pltpu.sync_copy(zerosd.at[pl.ds(0, WB)], deg_sh.at[pl.ds(sid * WB, WB)])

    @pl.when(sid == NS - 1)
    def _():
        pltpu.sync_copy(zerosd, deg_sh.at[pl.ds((NS - 1) * WB, NDEG - (NS - 1) * WB)])

    pltpu.sync_copy(ones8, ones_v)
    pltpu.sync_copy(dst1d.at[pl.ds(wid * EPW, EPW)], dsts_v.at[pl.ds(0, EPW)])
    for k in range((EPAD - EPW) // 16):
        dsts_v[pl.ds(EPW + 16 * k, 16)] = jnp.full((16,), DUMP, jnp.int32)
    plsc.subcore_barrier()

    def sc_desc(c):
        return pltpu.make_async_copy(
            ones_v, deg_sh.at[dsts_v.at[pl.ds(c * CHK_D, CHK_D)]], sem
        )

    def body(c, carry):
        pltpu.async_copy(
            ones_v, deg_sh.at[dsts_v.at[pl.ds(c * CHK_D, CHK_D)]], sem, add=True
        )

        @pl.when(c >= DEG_W)
        def _():
            sc_desc(c).wait()

        return carry

    lax.fori_loop(0, NCHK_D, body, 0)

    def drain(c, carry):
        sc_desc(c).wait()
        return carry

    lax.fori_loop(0, DEG_W, drain, 0)
    plsc.subcore_barrier()

    @pl.when(sid < NS - 1)
    def _():
        pltpu.sync_copy(
            deg_sh.at[pl.ds(sid * WB, WB)], degp.at[cid, pl.ds(sid * WB, WB)]
        )

    @pl.when(sid == NS - 1)
    def _():
        pltpu.sync_copy(
            deg_sh.at[pl.ds((NS - 1) * WB, WB_LAST)],
            degp.at[cid, pl.ds((NS - 1) * WB, WB_LAST)],
        )


def _sc_calls():
    mesh = plsc.VectorSubcoreMesh(
        core_axis_name="c", subcore_axis_name="s", num_cores=NC, num_subcores=NS
    )
    deg_call = pl.kernel(
        _deg_body,
        out_type=jax.ShapeDtypeStruct((NC, N_NODES, DDEG), jnp.float32),
        mesh=mesh,
        scratch_types=[
            pltpu.VMEM((CHK_D, DDEG), jnp.float32),
            pltpu.VMEM((EPAD,), jnp.int32),
            pltpu.VMEM_SHARED((NDEG, DDEG), jnp.float32),
            pltpu.SemaphoreType.DMA,
        ],
    )
    msg_call = pl.kernel(
        _msg_body,
        out_type=jax.ShapeDtypeStruct((NC, N_NODES, CH), jnp.float32),
        mesh=mesh,
        scratch_types=[
            pltpu.VMEM((EPW,), jnp.int32),
            pltpu.VMEM((EPW,), jnp.int32),
            pltpu.VMEM((NBUF, CHK_M, CH), jnp.float32),
            pltpu.VMEM_SHARED((N_NODES, CH), jnp.float32),
        ]
        + [pltpu.SemaphoreType.DMA] * (2 * NBUF),
    )
    return deg_call, msg_call


CHK_M = 80
NCHK_M = EPW // CHK_M
NBUF = 3
PRO = NCHK_M % NBUF
NROUND = (NCHK_M - PRO) // NBUF


def _msg_body(
    src1d, dst1d, y, zerosz, zpart, srcs_v, dsts_v, rows_v, z_sh,
    g0, g1, g2, s0, s1, s2
):
    gsem = [g0, g1, g2]
    ssem = [s0, s1, s2]
    cid = lax.axis_index("c")
    sid = lax.axis_index("s")
    wid = sid * NC + cid

    @pl.when(sid < NS - 1)
    def _():
        pltpu.async_copy(
            zerosz.at[pl.ds(0, WB)], z_sh.at[pl.ds(sid * WB, WB)], g0
        )

    @pl.when(sid == NS - 1)
    def _():
        pltpu.async_copy(
            zerosz.at[pl.ds(0, WB)], z_sh.at[pl.ds((NS - 1) * WB, WB)], g0
        )
        pltpu.async_copy(
            zerosz.at[pl.ds(WB, WB_LAST - WB)],
            z_sh.at[pl.ds((NS - 1) * WB + WB, WB_LAST - WB)],
            s0,
        )
        pltpu.make_async_copy(
            zerosz.at[pl.ds(WB, WB_LAST - WB)],
            z_sh.at[pl.ds((NS - 1) * WB + WB, WB_LAST - WB)],
            s0,
        ).wait()

    d_src = pltpu.async_copy(src1d.at[pl.ds(wid * EPW, EPW)], srcs_v, g1)
    d_dst = pltpu.async_copy(dst1d.at[pl.ds(wid * EPW, EPW)], dsts_v, g2)
    pltpu.make_async_copy(
        zerosz.at[pl.ds(0, WB)], z_sh.at[pl.ds(sid * WB, WB)], g0
    ).wait()
    d_src.wait()
    d_dst.wait()
    plsc.subcore_barrier()

    def g_desc(c, b):
        return pltpu.make_async_copy(
            y.at[srcs_v.at[pl.ds(c * CHK_M, CHK_M)]], rows_v.at[b], gsem[b]
        )

    for c in range(PRO):
        g_desc(c, 0).start()
        g_desc(c, 0).wait()
        pltpu.sync_copy(
            rows_v.at[0], z_sh.at[dsts_v.at[pl.ds(c * CHK_M, CHK_M)]], add=True
        )

    for b in range(NBUF):
        g_desc(PRO + b, b).start()

    def round_(i, carry):
        c0 = PRO + i * NBUF
        sdescs = []
        for b in range(NBUF):
            g_desc(c0 + b, b).wait()
            sdescs.append(
                pltpu.async_copy(
                    rows_v.at[b],
                    z_sh.at[dsts_v.at[pl.ds((c0 + b) * CHK_M, CHK_M)]],
                    ssem[b],
                    add=True,
                )
            )
        for b in range(NBUF):
            sdescs[b].wait()

            @pl.when(i < NROUND - 1)
            def _():
                g_desc(c0 + NBUF + b, b).start()

        return carry

    lax.fori_loop(0, NROUND, round_, 0)
    plsc.subcore_barrier()

    @pl.when(sid < NS - 1)
    def _():
        pltpu.sync_copy(
            z_sh.at[pl.ds(sid * WB, WB)], zpart.at[cid, pl.ds(sid * WB, WB)]
        )

    @pl.when(sid == NS - 1)
    def _():
        pltpu.sync_copy(
            z_sh.at[pl.ds((NS - 1) * WB, WB_LAST)],
            zpart.at[cid, pl.ds((NS - 1) * WB, WB_LAST)],
        )


def _mm_body(x_ref, w_ref, degp_ref, y_ref):
    deg = 1.0 + degp_ref[0, :, 0] + degp_ref[1, :, 0]
    dinv = lax.rsqrt(deg)
    xw = jnp.dot(x_ref[...], w_ref[...], preferred_element_type=jnp.float32)
    y_ref[...] = xw * dinv[:, None]


def _mm_call(x, w, degp):
    return pl.pallas_call(
        _mm_body,
        grid=(NBLK,),
        in_specs=[
            pl.BlockSpec((RBLK, CH), lambda i: (i, 0)),
            pl.BlockSpec((CH, CH), lambda i: (0, 0)),
            pl.BlockSpec((NC, RBLK, DDEG), lambda i: (0, i, 0)),
        ],
        out_specs=pl.BlockSpec((RBLK, CH), lambda i: (i, 0)),
        out_shape=jax.ShapeDtypeStruct((N_NODES, CH), jnp.float32),
    )(x, w, degp)


def _fin_body(
    zp_ref, y_ref, degp_ref, bconv_ref, x0_ref, w0_ref, b0_ref,
    w1_ref, b1_ref, w2_ref, b2_ref, out_ref, acc_ref
):
    i = pl.program_id(0)

    @pl.when(i == 0)
    def _():
        acc_ref[...] = jnp.zeros_like(acc_ref)

    deg = 1.0 + degp_ref[0, :, 0] + degp_ref[1, :, 0]
    dinv = lax.rsqrt(deg)
    t = (zp_ref[0] + zp_ref[1] + y_ref[...]) * dinv[:, None] + bconv_ref[...]
    t = jnp.maximum(t, 0.0)
    bm = jnp.max(t, axis=0, keepdims=True)
    acc_ref[...] = jnp.maximum(acc_ref[...], bm)

    @pl.when(i == pl.num_programs(0) - 1)
    def _():
        hmax = acc_ref[...]
        news = jnp.dot(x0_ref[...], w0_ref[...], preferred_element_type=jnp.float32)
        news = jnp.maximum(news + b0_ref[...], 0.0)
        cat = jnp.concatenate([news, hmax], axis=1)
        u = jnp.dot(cat, w1_ref[...], preferred_element_type=jnp.float32)
        u = jnp.maximum(u + b1_ref[...], 0.0)
        out_ref[...] = (
            jnp.dot(u, w2_ref[...], preferred_element_type=jnp.float32) + b2_ref[...]
        )


def _fin_call(zp, y, degp, bconv, x0, w0, b0, w1, b1, w2, b2):
    full = lambda shape: pl.BlockSpec(shape, lambda i: tuple(0 for _ in shape))
    return pl.pallas_call(
        _fin_body,
        grid=(NBLK,),
        in_specs=[
            pl.BlockSpec((NC, RBLK, CH), lambda i: (0, i, 0)),
            pl.BlockSpec((RBLK, CH), lambda i: (i, 0)),
            pl.BlockSpec((NC, RBLK, DDEG), lambda i: (0, i, 0)),
            full((1, CH)),
            full((1, CH)),
            full((CH, CH)),
            full((1, CH)),
            full((2 * CH, CH)),
            full((1, CH)),
            full((CH, CH)),
            full((1, CH)),
        ],
        out_specs=pl.BlockSpec((1, CH), lambda i: (0, 0)),
        out_shape=jax.ShapeDtypeStruct((1, CH), jnp.float32),
        scratch_shapes=[pltpu.VMEM((1, CH), jnp.float32)],
    )(zp, y, degp, bconv, x0, w0, b0, w1, b1, w2, b2)


def kernel(x, edge_index, W_conv, b_conv, lin0_W, lin0_b, lin1_W, lin1_b, lin2_W, lin2_b):
    src1d = edge_index[0]
    dst1d = edge_index[1]
    zerosd = jnp.zeros((NDEG - (NS - 1) * WB, DDEG), jnp.float32)
    ones8 = jnp.ones((CHK_D, DDEG), jnp.float32)
    zerosz = jnp.zeros((WB_LAST, CH), jnp.float32)

    deg_call, msg_call = _sc_calls()
    degp = deg_call(dst1d, zerosd, ones8)
    y = _mm_call(x, W_conv, degp)
    zp = msg_call(src1d, dst1d, y, zerosz)
    out = _fin_call(
        zp, y, degp,
        b_conv.reshape(1, CH), x[0:1], lin0_W, lin0_b.reshape(1, CH),
        lin1_W, lin1_b.reshape(1, CH), lin2_W, lin2_b.reshape(1, CH),
    )
    return out.reshape(CH)


# speedup vs baseline: 1.0456x; 1.0456x over previous
import jax
import jax.numpy as jnp
from jax import lax
from jax.experimental import pallas as pl
from jax.experimental.pallas import tpu as pltpu
from jax.experimental.pallas import tpu_sc as plsc

N_NODES = 10000
E_EDGES = 320000
CH = 128

NC = 2
NS = 16
NW = NC * NS
EPW = E_EDGES // NW
RPT = N_NODES // NS
WB = 624
WB_LAST = N_NODES - 15 * WB
DDEG = 8

CHK_D = 128
EPAD = 10112
NCHK_D = EPAD // CHK_D
DUMP = N_NODES
NDEG = N_NODES + 8

RBLK = 1000
NBLK = N_NODES // RBLK

DEG_W = 8


def _deg_body(dst1d, zerosd, ones8, degp, ones_v, dsts_v, deg_sh, sem):
    cid = lax.axis_index("c")
    sid = lax.axis_index("s")
    wid = sid * NC + cid

    @pl.when(sid < NS - 1)
    def _():
        pltpu.sync_copy(zerosd.at[pl.ds(0, WB)], deg_sh.at[pl.ds(sid * WB, WB)])

    @pl.when(sid == NS - 1)
    def _():
        pltpu.sync_copy(zerosd, deg_sh.at[pl.ds((NS - 1) * WB, NDEG - (NS - 1) * WB)])

    pltpu.sync_copy(ones8, ones_v)
    pltpu.sync_copy(dst1d.at[pl.ds(wid * EPW, EPW)], dsts_v.at[pl.ds(0, EPW)])
    for k in range((EPAD - EPW) // 16):
        dsts_v[pl.ds(EPW + 16 * k, 16)] = jnp.full((16,), DUMP, jnp.int32)
    plsc.subcore_barrier()

    def sc_desc(c):
        return pltpu.make_async_copy(
            ones_v, deg_sh.at[dsts_v.at[pl.ds(c * CHK_D, CHK_D)]], sem
        )

    def body(c, carry):
        pltpu.async_copy(
            ones_v, deg_sh.at[dsts_v.at[pl.ds(c * CHK_D, CHK_D)]], sem, add=True
        )

        @pl.when(c >= DEG_W)
        def _():
            sc_desc(c).wait()

        return carry

    lax.fori_loop(0, NCHK_D, body, 0)

    def drain(c, carry):
        sc_desc(c).wait()
        return carry

    lax.fori_loop(0, DEG_W, drain, 0)
    plsc.subcore_barrier()

    @pl.when(sid < NS - 1)
    def _():
        pltpu.sync_copy(
            deg_sh.at[pl.ds(sid * WB, WB)], degp.at[cid, pl.ds(sid * WB, WB)]
        )

    @pl.when(sid == NS - 1)
    def _():
        pltpu.sync_copy(
            deg_sh.at[pl.ds((NS - 1) * WB, WB_LAST)],
            degp.at[cid, pl.ds((NS - 1) * WB, WB_LAST)],
        )


def _sc_calls():
    mesh = plsc.VectorSubcoreMesh(
        core_axis_name="c", subcore_axis_name="s", num_cores=NC, num_subcores=NS
    )
    deg_call = pl.kernel(
        _deg_body,
        out_type=jax.ShapeDtypeStruct((NC, N_NODES, DDEG), jnp.float32),
        mesh=mesh,
        scratch_types=[
            pltpu.VMEM((CHK_D, DDEG), jnp.float32),
            pltpu.VMEM((EPAD,), jnp.int32),
            pltpu.VMEM_SHARED((NDEG, DDEG), jnp.float32),
            pltpu.SemaphoreType.DMA,
        ],
    )
    msg_call = pl.kernel(
        _msg_body,
        out_type=jax.ShapeDtypeStruct((NC, N_NODES, CH), jnp.float32),
        mesh=mesh,
        scratch_types=[
            pltpu.VMEM((EPW,), jnp.int32),
            pltpu.VMEM((EPW,), jnp.int32),
            pltpu.VMEM((NBUF, CHK_M, CH), jnp.float32),
            pltpu.VMEM_SHARED((N_NODES, CH), jnp.float32),
        ]
        + [pltpu.SemaphoreType.DMA] * (2 * NBUF),
    )
    return deg_call, msg_call


CHK_M = 40
NCHK_M = EPW // CHK_M
NBUF = 5
PRO = NCHK_M % NBUF
NROUND = (NCHK_M - PRO) // NBUF


def _msg_body(
    src1d, dst1d, y, zerosz, zpart, srcs_v, dsts_v, rows_v, z_sh,
    g0, g1, g2, g3, g4, s0, s1, s2, s3, s4
):
    gsem = [g0, g1, g2, g3, g4]
    ssem = [s0, s1, s2, s3, s4]
    cid = lax.axis_index("c")
    sid = lax.axis_index("s")
    wid = sid * NC + cid

    @pl.when(sid < NS - 1)
    def _():
        pltpu.async_copy(
            zerosz.at[pl.ds(0, WB)], z_sh.at[pl.ds(sid * WB, WB)], g0
        )

    @pl.when(sid == NS - 1)
    def _():
        pltpu.async_copy(
            zerosz.at[pl.ds(0, WB)], z_sh.at[pl.ds((NS - 1) * WB, WB)], g0
        )
        pltpu.async_copy(
            zerosz.at[pl.ds(WB, WB_LAST - WB)],
            z_sh.at[pl.ds((NS - 1) * WB + WB, WB_LAST - WB)],
            s0,
        )
        pltpu.make_async_copy(
            zerosz.at[pl.ds(WB, WB_LAST - WB)],
            z_sh.at[pl.ds((NS - 1) * WB + WB, WB_LAST - WB)],
            s0,
        ).wait()

    d_src = pltpu.async_copy(src1d.at[pl.ds(wid * EPW, EPW)], srcs_v, g1)
    d_dst = pltpu.async_copy(dst1d.at[pl.ds(wid * EPW, EPW)], dsts_v, g2)
    pltpu.make_async_copy(
        zerosz.at[pl.ds(0, WB)], z_sh.at[pl.ds(sid * WB, WB)], g0
    ).wait()
    d_src.wait()
    d_dst.wait()
    plsc.subcore_barrier()

    def g_desc(c, b):
        return pltpu.make_async_copy(
            y.at[srcs_v.at[pl.ds(c * CHK_M, CHK_M)]], rows_v.at[b], gsem[b]
        )

    for c in range(PRO):
        g_desc(c, 0).start()
        g_desc(c, 0).wait()
        pltpu.sync_copy(
            rows_v.at[0], z_sh.at[dsts_v.at[pl.ds(c * CHK_M, CHK_M)]], add=True
        )

    for b in range(NBUF):
        g_desc(PRO + b, b).start()

    def round_(i, carry):
        c0 = PRO + i * NBUF
        sdescs = []
        for b in range(NBUF):
            g_desc(c0 + b, b).wait()
            sdescs.append(
                pltpu.async_copy(
                    rows_v.at[b],
                    z_sh.at[dsts_v.at[pl.ds((c0 + b) * CHK_M, CHK_M)]],
                    ssem[b],
                    add=True,
                )
            )
        for b in range(NBUF):
            sdescs[b].wait()

            @pl.when(i < NROUND - 1)
            def _():
                g_desc(c0 + NBUF + b, b).start()

        return carry

    lax.fori_loop(0, NROUND, round_, 0)
    plsc.subcore_barrier()

    @pl.when(sid < NS - 1)
    def _():
        pltpu.sync_copy(
            z_sh.at[pl.ds(sid * WB, WB)], zpart.at[cid, pl.ds(sid * WB, WB)]
        )

    @pl.when(sid == NS - 1)
    def _():
        pltpu.sync_copy(
            z_sh.at[pl.ds((NS - 1) * WB, WB_LAST)],
            zpart.at[cid, pl.ds((NS - 1) * WB, WB_LAST)],
        )


def _mm_body(x_ref, w_ref, degp_ref, y_ref):
    deg = 1.0 + degp_ref[0, :, 0] + degp_ref[1, :, 0]
    dinv = lax.rsqrt(deg)
    xw = jnp.dot(x_ref[...], w_ref[...], preferred_element_type=jnp.float32)
    y_ref[...] = xw * dinv[:, None]


def _mm_call(x, w, degp):
    return pl.pallas_call(
        _mm_body,
        grid=(NBLK,),
        in_specs=[
            pl.BlockSpec((RBLK, CH), lambda i: (i, 0)),
            pl.BlockSpec((CH, CH), lambda i: (0, 0)),
            pl.BlockSpec((NC, RBLK, DDEG), lambda i: (0, i, 0)),
        ],
        out_specs=pl.BlockSpec((RBLK, CH), lambda i: (i, 0)),
        out_shape=jax.ShapeDtypeStruct((N_NODES, CH), jnp.float32),
    )(x, w, degp)


def _fin_body(
    zp_ref, y_ref, degp_ref, bconv_ref, x0_ref, w0_ref, b0_ref,
    w1_ref, b1_ref, w2_ref, b2_ref, out_ref, acc_ref
):
    i = pl.program_id(0)

    @pl.when(i == 0)
    def _():
        acc_ref[...] = jnp.zeros_like(acc_ref)

    deg = 1.0 + degp_ref[0, :, 0] + degp_ref[1, :, 0]
    dinv = lax.rsqrt(deg)
    t = (zp_ref[0] + zp_ref[1] + y_ref[...]) * dinv[:, None] + bconv_ref[...]
    t = jnp.maximum(t, 0.0)
    bm = jnp.max(t, axis=0, keepdims=True)
    acc_ref[...] = jnp.maximum(acc_ref[...], bm)

    @pl.when(i == pl.num_programs(0) - 1)
    def _():
        hmax = acc_ref[...]
        news = jnp.dot(x0_ref[...], w0_ref[...], preferred_element_type=jnp.float32)
        news = jnp.maximum(news + b0_ref[...], 0.0)
        cat = jnp.concatenate([news, hmax], axis=1)
        u = jnp.dot(cat, w1_ref[...], preferred_element_type=jnp.float32)
        u = jnp.maximum(u + b1_ref[...], 0.0)
        out_ref[...] = (
            jnp.dot(u, w2_ref[...], preferred_element_type=jnp.float32) + b2_ref[...]
        )


def _fin_call(zp, y, degp, bconv, x0, w0, b0, w1, b1, w2, b2):
    full = lambda shape: pl.BlockSpec(shape, lambda i: tuple(0 for _ in shape))
    return pl.pallas_call(
        _fin_body,
        grid=(NBLK,),
        in_specs=[
            pl.BlockSpec((NC, RBLK, CH), lambda i: (0, i, 0)),
            pl.BlockSpec((RBLK, CH), lambda i: (i, 0)),
            pl.BlockSpec((NC, RBLK, DDEG), lambda i: (0, i, 0)),
            full((1, CH)),
            full((1, CH)),
            full((CH, CH)),
            full((1, CH)),
            full((2 * CH, CH)),
            full((1, CH)),
            full((CH, CH)),
            full((1, CH)),
        ],
        out_specs=pl.BlockSpec((1, CH), lambda i: (0, 0)),
        out_shape=jax.ShapeDtypeStruct((1, CH), jnp.float32),
        scratch_shapes=[pltpu.VMEM((1, CH), jnp.float32)],
    )(zp, y, degp, bconv, x0, w0, b0, w1, b1, w2, b2)


def kernel(x, edge_index, W_conv, b_conv, lin0_W, lin0_b, lin1_W, lin1_b, lin2_W, lin2_b):
    src1d = edge_index[0]
    dst1d = edge_index[1]
    zerosd = jnp.zeros((NDEG - (NS - 1) * WB, DDEG), jnp.float32)
    ones8 = jnp.ones((CHK_D, DDEG), jnp.float32)
    zerosz = jnp.zeros((WB_LAST, CH), jnp.float32)

    deg_call, msg_call = _sc_calls()
    degp = deg_call(dst1d, zerosd, ones8)
    y = _mm_call(x, W_conv, degp)
    zp = msg_call(src1d, dst1d, y, zerosz)
    out = _fin_call(
        zp, y, degp,
        b_conv.reshape(1, CH), x[0:1], lin0_W, lin0_b.reshape(1, CH),
        lin1_W, lin1_b.reshape(1, CH), lin2_W, lin2_b.reshape(1, CH),
    )
    return out.reshape(CH)
